# baseline (speedup 1.0000x reference)
import jax
import jax.numpy as jnp
from jax.experimental import pallas as pl

_SELF_BIG = 1e9
_SEL_BIG = 2e9
_NEG_BIG = -1e30
_K = 20


def _hst_body(x_ref, p_ref, w_ref, asrc_ref, adst_ref, h_ref, s_ref, t_ref,
              pn_ref):
    x = x_ref[0]
    h = jnp.dot(x, w_ref[...], preferred_element_type=jnp.float32)
    h_ref[0] = h.astype(jnp.bfloat16)
    s_ref[0] = jnp.dot(h, asrc_ref[...], preferred_element_type=jnp.float32)
    t_ref[0] = jnp.dot(h, adst_ref[...], preferred_element_type=jnp.float32)
    p = p_ref[0]
    pn_ref[0] = jnp.sum(p * p, axis=1, keepdims=True)


def _hst(x, p, w, asrc, adst):
    b, n, cin = x.shape
    c = w.shape[1]
    rh = 1024 if n % 1024 == 0 else n
    return pl.pallas_call(
        _hst_body,
        grid=(b, n // rh),
        in_specs=[
            pl.BlockSpec((1, rh, cin), lambda i, j: (i, j, 0)),
            pl.BlockSpec((1, rh, 3), lambda i, j: (i, j, 0)),
            pl.BlockSpec((cin, c), lambda i, j: (0, 0)),
            pl.BlockSpec((c, 1), lambda i, j: (0, 0)),
            pl.BlockSpec((c, 1), lambda i, j: (0, 0)),
        ],
        out_specs=[
            pl.BlockSpec((1, rh, c), lambda i, j: (i, j, 0)),
            pl.BlockSpec((1, rh, 1), lambda i, j: (i, j, 0)),
            pl.BlockSpec((1, rh, 1), lambda i, j: (i, j, 0)),
            pl.BlockSpec((1, rh, 1), lambda i, j: (i, j, 0)),
        ],
        out_shape=[
            jax.ShapeDtypeStruct((b, n, c), jnp.bfloat16),
            jax.ShapeDtypeStruct((b, n, 1), jnp.float32),
            jax.ShapeDtypeStruct((b, n, 1), jnp.float32),
            jax.ShapeDtypeStruct((b, n, 1), jnp.float32),
        ],
    )(x, p, w, asrc, adst)


def _gat_body(pm2_ref, pt_ref, p_ref, pn_ref, pnt_ref, h_ref, s_ref, t_ref,
              b_ref, f_ref, pp_ref, *, r, n, k):
    q = p_ref[0]
    h = h_ref[0]
    st = s_ref[0]
    tt = t_ref[0]
    d0 = jnp.dot(pm2_ref[0], pt_ref[0], preferred_element_type=jnp.float32)
    dist = (pn_ref[0] + pnt_ref[0]) + d0
    rows = jax.lax.broadcasted_iota(jnp.int32, (n, r), 0)
    cols = pl.program_id(1) * r + jax.lax.broadcasted_iota(jnp.int32, (n, r), 1)
    dist = jnp.where(rows == cols, _SELF_BIG, dist)

    if n >= 3072 and n % 96 == 0:
        nc = 96
        cs = n // nc
        d3 = dist.reshape(nc, cs, r)
        gs = []
        dcur = d3
        for i in range(4):
            g = jnp.min(dcur, axis=1)
            gs.append(g)
            if i < 3:
                dcur = jnp.where(dcur == g[:, None, :], _SEL_BIG, dcur)
        cur, q1, q2, q3 = gs
        tau = None
        for _ in range(k):
            tau = jnp.min(cur, axis=0, keepdims=True)
            pick = cur == tau
            cur = jnp.where(pick, q1, cur)
            q1 = jnp.where(pick, q2, q1)
            q2 = jnp.where(pick, q3, q2)
            q3 = jnp.where(pick, _SEL_BIG, q3)
        sel = dist <= tau
    else:
        for _ in range(k):
            m = jnp.min(dist, axis=0, keepdims=True)
            dist = jnp.where(dist == m, _SEL_BIG, dist)
        sel = dist == _SEL_BIG

    lin = st + tt
    lin = jnp.where(lin >= 0.0, lin, 0.2 * lin)
    z = jnp.where(sel, jnp.exp(lin), 0.0).astype(jnp.bfloat16)
    denom = jnp.sum(z.astype(jnp.float32), axis=0, keepdims=True)
    out = jax.lax.dot_general(z, h, (((0,), (0,)), ((), ())),
                              preferred_element_type=jnp.float32)
    out = out * (1.0 / denom).reshape(r, 1)

    pc = jax.lax.broadcasted_iota(jnp.int32, (r // 4, r), 1)
    pr = jax.lax.broadcasted_iota(jnp.int32, (r // 4, r), 0)
    pm = jnp.where(pc // 4 == pr, 0.25, 0.0)
    f_ref[0] = jnp.dot(pm, out, preferred_element_type=jnp.float32) + b_ref[...]
    pp_ref[0] = jnp.dot(pm, q, preferred_element_type=jnp.float32)


def _pick_r(n):
    if n == 12288:
        return 128
    if n == 3072:
        return 256
    if n <= 1024:
        return n
    r = 512
    while n % r != 0:
        r //= 2
    return r


def _gat(pm2, pt, p, pn, pnt, h, s, t, b):
    nb, n, c = h.shape
    r = _pick_r(n)
    import functools
    body = functools.partial(_gat_body, r=r, n=n, k=_K)
    return pl.pallas_call(
        body,
        grid=(nb, n // r),
        in_specs=[
            pl.BlockSpec((1, n, 3), lambda i, j: (i, 0, 0)),
            pl.BlockSpec((1, 3, r), lambda i, j: (i, 0, j)),
            pl.BlockSpec((1, r, 3), lambda i, j: (i, j, 0)),
            pl.BlockSpec((1, n, 1), lambda i, j: (i, 0, 0)),
            pl.BlockSpec((1, 1, r), lambda i, j: (i, 0, j)),
            pl.BlockSpec((1, n, c), lambda i, j: (i, 0, 0)),
            pl.BlockSpec((1, n, 1), lambda i, j: (i, 0, 0)),
            pl.BlockSpec((1, 1, r), lambda i, j: (i, 0, j)),
            pl.BlockSpec((1, c), lambda i, j: (0, 0)),
        ],
        out_specs=[
            pl.BlockSpec((1, r // 4, c), lambda i, j: (i, j, 0)),
            pl.BlockSpec((1, r // 4, 3), lambda i, j: (i, j, 0)),
        ],
        out_shape=[
            jax.ShapeDtypeStruct((nb, n // 4, c), jnp.float32),
            jax.ShapeDtypeStruct((nb, n // 4, 3), jnp.float32),
        ],
    )(pm2, pt, p, pn, pnt, h, s, t, b)


def kernel(x0, x1, keypointCoords0, keypointCoords1,
           W1, asrc1, adst1, b1, W2, asrc2, adst2, b2,
           W3, asrc3, adst3, b3, W4, asrc4, adst4, b4):
    params = [(W1, asrc1, adst1, b1), (W2, asrc2, adst2, b2),
              (W3, asrc3, adst3, b3), (W4, asrc4, adst4, b4)]
    f = jnp.concatenate([x0, x1], axis=0)
    p = jnp.concatenate([keypointCoords0, keypointCoords1], axis=0)
    for w, asrc, adst, b in params:
        n = f.shape[1]
        h, s, t, pn = _hst(f, p, w, asrc.reshape(-1, 1), adst.reshape(-1, 1))
        t = t.reshape(2, 1, n)
        pm2 = -2.0 * p
        pt = jnp.transpose(p, (0, 2, 1))
        pnt = pn.reshape(2, 1, n)
        f, p = _gat(pm2, pt, p, pn, pnt, h, s, t, b.reshape(1, -1))
    return jnp.concatenate([f[0], f[1]], axis=0)
